# Initial kernel scaffold; baseline (speedup 1.0000x reference)
import jax
import jax.numpy as jnp
from jax.experimental import pallas as pl


def kernel(x, edge_index, W_init, W_root, bias):
    raise NotImplementedError("write your pallas kernel here")


# speedup vs baseline: 28.7143x; 28.7143x over previous
import functools

import jax
import jax.numpy as jnp
from jax import lax
from jax.experimental import pallas as pl
from jax.experimental.pallas import tpu as pltpu
from jax.experimental.pallas import tpu_sc as plsc

N = 10000
E = 320000
F_IN = 128
F_OUT = 64

NC = 2
NS = 16
NW = NC * NS
EPW = E // NW
CH = 125
NCHUNK = EPW // CH
SPAN = 632
N_PAD = SPAN * NS
DEG_PAD = 10240
DEG_SPAN = DEG_PAD // NS
ZROWS = 80

_MESH = plsc.VectorSubcoreMesh(core_axis_name="c", subcore_axis_name="s")
_SC_PARAMS = pltpu.CompilerParams(use_tc_tiling_on_sc=False)


DEG_W = 8


@functools.partial(
    pl.kernel,
    out_type=jax.ShapeDtypeStruct((NC, DEG_PAD, DEG_W), jnp.float32),
    mesh=_MESH,
    compiler_params=_SC_PARAMS,
    scratch_types=[
        pltpu.VMEM((NCHUNK, CH), jnp.int32),
        pltpu.VMEM((CH, DEG_W), jnp.float32),
        pltpu.VMEM((DEG_SPAN, DEG_W), jnp.float32),
        pltpu.VMEM_SHARED((DEG_PAD, DEG_W), jnp.float32),
    ],
)
def _deg_kernel(dst3, ones8, zeros8, degp, dst2, ones_v, zb, deg_sh):
    cid = lax.axis_index("c")
    sid = lax.axis_index("s")
    wid = cid * NS + sid

    pltpu.sync_copy(zeros8, zb)
    pltpu.sync_copy(ones8, ones_v)
    pltpu.sync_copy(zb, deg_sh.at[pl.ds(sid * DEG_SPAN, DEG_SPAN)])
    pltpu.sync_copy(dst3.at[wid], dst2)
    plsc.subcore_barrier()

    @pl.loop(0, NCHUNK)
    def _accum(j):
        pltpu.sync_copy(ones_v, deg_sh.at[dst2.at[j]], add=True)

    plsc.subcore_barrier()
    pltpu.sync_copy(
        deg_sh.at[pl.ds(sid * DEG_SPAN, DEG_SPAN)],
        degp.at[cid, pl.ds(sid * DEG_SPAN, DEG_SPAN)],
    )


@functools.partial(
    pl.kernel,
    out_type=jax.ShapeDtypeStruct((NC, N_PAD, F_OUT), jnp.float32),
    mesh=_MESH,
    compiler_params=_SC_PARAMS,
    scratch_types=[
        pltpu.VMEM((NCHUNK, CH), jnp.int32),
        pltpu.VMEM((NCHUNK, CH), jnp.int32),
        pltpu.VMEM((CH, F_OUT), jnp.float32),
        pltpu.VMEM((CH, F_OUT), jnp.float32),
        pltpu.VMEM((ZROWS, F_OUT), jnp.float32),
        pltpu.VMEM_SHARED((N_PAD, F_OUT), jnp.float32),
        pltpu.SemaphoreType.DMA,
        pltpu.SemaphoreType.DMA,
    ],
)
def _agg_kernel(hp, src3, dst3, out, src2, dst2, buf0, buf1, zbuf, agg_sh, sem0, sem1):
    cid = lax.axis_index("c")
    sid = lax.axis_index("s")
    wid = cid * NS + sid

    @pl.loop(0, ZROWS)
    def _zero(r):
        for c in range(F_OUT // 16):
            zbuf[r, pl.ds(c * 16, 16)] = jnp.zeros((16,), jnp.float32)

    r0 = sid * SPAN
    for t in range(SPAN // ZROWS):
        pltpu.sync_copy(zbuf, agg_sh.at[pl.ds(r0 + t * ZROWS, ZROWS)])
    rem = SPAN - (SPAN // ZROWS) * ZROWS
    pltpu.sync_copy(
        zbuf.at[pl.ds(0, rem)],
        agg_sh.at[pl.ds(r0 + SPAN - rem, rem)],
    )
    pltpu.sync_copy(src3.at[wid], src2)
    pltpu.sync_copy(dst3.at[wid], dst2)
    plsc.subcore_barrier()

    @pl.loop(0, NCHUNK)
    def _edges(j):
        pltpu.sync_copy(hp.at[src2.at[j]], buf0)
        pltpu.sync_copy(buf0, agg_sh.at[dst2.at[j]], add=True)

    plsc.subcore_barrier()
    pltpu.sync_copy(agg_sh.at[pl.ds(r0, SPAN)], out.at[cid, pl.ds(r0, SPAN)])


_RB = 1000


def _prep_body(x_ref, wi_ref, wr_ref, b_ref, d0_ref, d1_ref,
               hp_ref, rootb_ref, dinv_ref):
    x = x_ref[...]
    deg = d0_ref[...] + d1_ref[...]
    dinv = jnp.where(deg > 0, lax.rsqrt(deg), 0.0)
    h = jnp.dot(x, wi_ref[...], preferred_element_type=jnp.float32)
    hp_ref[...] = h * dinv
    rootb_ref[...] = (
        jnp.dot(x, wr_ref[...], preferred_element_type=jnp.float32) + b_ref[...]
    )
    dinv_ref[...] = dinv


def _prep(x, wi, wr, b2, d0, d1):
    grid = (N // _RB,)
    return pl.pallas_call(
        _prep_body,
        grid=grid,
        in_specs=[
            pl.BlockSpec((_RB, F_IN), lambda i: (i, 0)),
            pl.BlockSpec((F_IN, F_OUT), lambda i: (0, 0)),
            pl.BlockSpec((F_IN, F_OUT), lambda i: (0, 0)),
            pl.BlockSpec((1, F_OUT), lambda i: (0, 0)),
            pl.BlockSpec((_RB, 1), lambda i: (i, 0)),
            pl.BlockSpec((_RB, 1), lambda i: (i, 0)),
        ],
        out_specs=[
            pl.BlockSpec((_RB, F_OUT), lambda i: (i, 0)),
            pl.BlockSpec((_RB, F_OUT), lambda i: (i, 0)),
            pl.BlockSpec((_RB, 1), lambda i: (i, 0)),
        ],
        out_shape=[
            jax.ShapeDtypeStruct((N, F_OUT), jnp.float32),
            jax.ShapeDtypeStruct((N, F_OUT), jnp.float32),
            jax.ShapeDtypeStruct((N, 1), jnp.float32),
        ],
    )(x, wi, wr, b2, d0, d1)


def _final_body(p0_ref, p1_ref, dinv_ref, rootb_ref, o_ref):
    agg = p0_ref[...] + p1_ref[...]
    o_ref[...] = jnp.maximum(dinv_ref[...] * agg + rootb_ref[...], 0.0)


def _final(p0, p1, dinv, rootb):
    grid = (N // _RB,)
    return pl.pallas_call(
        _final_body,
        grid=grid,
        in_specs=[
            pl.BlockSpec((_RB, F_OUT), lambda i: (i, 0)),
            pl.BlockSpec((_RB, F_OUT), lambda i: (i, 0)),
            pl.BlockSpec((_RB, 1), lambda i: (i, 0)),
            pl.BlockSpec((_RB, F_OUT), lambda i: (i, 0)),
        ],
        out_specs=pl.BlockSpec((_RB, F_OUT), lambda i: (i, 0)),
        out_shape=jax.ShapeDtypeStruct((N, F_OUT), jnp.float32),
    )(p0, p1, dinv, rootb)


def kernel(x, edge_index, W_init, W_root, bias):
    src3 = edge_index[0].reshape(NW, NCHUNK, CH)
    dst3 = edge_index[1].reshape(NW, NCHUNK, CH)
    ones8 = jnp.ones((CH, DEG_W), jnp.float32)
    zeros8 = jnp.zeros((DEG_SPAN, DEG_W), jnp.float32)
    degp = _deg_kernel(dst3, ones8, zeros8)
    d0 = degp[0, :N, 0].reshape(N, 1)
    d1 = degp[1, :N, 0].reshape(N, 1)
    hp, rootb, dinv = _prep(x, W_init, W_root, bias.reshape(1, F_OUT), d0, d1)
    p = _agg_kernel(hp, src3, dst3)
    return _final(p[0, :N], p[1, :N], dinv, rootb)


# speedup vs baseline: 38.8548x; 1.3532x over previous
import functools

import jax
import jax.numpy as jnp
from jax import lax
from jax.experimental import pallas as pl
from jax.experimental.pallas import tpu as pltpu
from jax.experimental.pallas import tpu_sc as plsc

N = 10000
E = 320000
F_IN = 128
F_OUT = 64

NC = 2
NS = 16
NW = NC * NS
EPW = E // NW
CH = 125
NCHUNK = EPW // CH
SPAN = 632
N_PAD = SPAN * NS
DEG_PAD = 10240
DEG_SPAN = DEG_PAD // NS
ZROWS = 80

_MESH = plsc.VectorSubcoreMesh(core_axis_name="c", subcore_axis_name="s")
_SC_PARAMS = pltpu.CompilerParams(use_tc_tiling_on_sc=False)


DEG_W = 8


@functools.partial(
    pl.kernel,
    out_type=jax.ShapeDtypeStruct((NC, DEG_PAD, DEG_W), jnp.float32),
    mesh=_MESH,
    compiler_params=_SC_PARAMS,
    scratch_types=[
        pltpu.VMEM((NCHUNK, CH), jnp.int32),
        pltpu.VMEM((CH, DEG_W), jnp.float32),
        pltpu.VMEM((DEG_SPAN, DEG_W), jnp.float32),
        pltpu.VMEM_SHARED((DEG_PAD, DEG_W), jnp.float32),
    ],
)
def _deg_kernel(dst3, ones8, zeros8, degp, dst2, ones_v, zb, deg_sh):
    cid = lax.axis_index("c")
    sid = lax.axis_index("s")
    wid = cid * NS + sid

    pltpu.sync_copy(zeros8, zb)
    pltpu.sync_copy(ones8, ones_v)
    pltpu.sync_copy(zb, deg_sh.at[pl.ds(sid * DEG_SPAN, DEG_SPAN)])
    pltpu.sync_copy(dst3.at[wid], dst2)
    plsc.subcore_barrier()

    @pl.loop(0, NCHUNK)
    def _accum(j):
        pltpu.sync_copy(ones_v, deg_sh.at[dst2.at[j]], add=True)

    plsc.subcore_barrier()
    pltpu.sync_copy(
        deg_sh.at[pl.ds(sid * DEG_SPAN, DEG_SPAN)],
        degp.at[cid, pl.ds(sid * DEG_SPAN, DEG_SPAN)],
    )


@functools.partial(
    pl.kernel,
    out_type=jax.ShapeDtypeStruct((NC, N_PAD, F_OUT), jnp.float32),
    mesh=_MESH,
    compiler_params=_SC_PARAMS,
    scratch_types=[
        pltpu.VMEM((NCHUNK, CH), jnp.int32),
        pltpu.VMEM((NCHUNK, CH), jnp.int32),
        pltpu.VMEM((CH, F_OUT), jnp.float32),
        pltpu.VMEM((CH, F_OUT), jnp.float32),
        pltpu.VMEM((ZROWS, F_OUT), jnp.float32),
        pltpu.VMEM_SHARED((N_PAD, F_OUT), jnp.float32),
        pltpu.SemaphoreType.DMA,
        pltpu.SemaphoreType.DMA,
    ],
)
def _agg_kernel(hp, src3, dst3, out, src2, dst2, buf0, buf1, zbuf, agg_sh, sem0, sem1):
    cid = lax.axis_index("c")
    sid = lax.axis_index("s")
    wid = cid * NS + sid

    @pl.loop(0, ZROWS)
    def _zero(r):
        for c in range(F_OUT // 16):
            zbuf[r, pl.ds(c * 16, 16)] = jnp.zeros((16,), jnp.float32)

    r0 = sid * SPAN
    for t in range(SPAN // ZROWS):
        pltpu.sync_copy(zbuf, agg_sh.at[pl.ds(r0 + t * ZROWS, ZROWS)])
    rem = SPAN - (SPAN // ZROWS) * ZROWS
    pltpu.sync_copy(
        zbuf.at[pl.ds(0, rem)],
        agg_sh.at[pl.ds(r0 + SPAN - rem, rem)],
    )
    pltpu.sync_copy(src3.at[wid], src2)
    pltpu.sync_copy(dst3.at[wid], dst2)
    plsc.subcore_barrier()

    pltpu.async_copy(hp.at[src2.at[0]], buf0, sem0)
    pltpu.async_copy(hp.at[src2.at[1]], buf1, sem1)

    @pl.loop(0, NCHUNK, step=2)
    def _edges(j):
        for b, (buf, sem) in enumerate(((buf0, sem0), (buf1, sem1))):
            i = j + b
            pltpu.make_async_copy(hp.at[src2.at[i]], buf, sem).wait()
            pltpu.sync_copy(buf, agg_sh.at[dst2.at[i]], add=True)
            nxt = jnp.minimum(i + 2, NCHUNK - 1)
            pltpu.async_copy(hp.at[src2.at[nxt]], buf, sem)

    pltpu.make_async_copy(hp.at[src2.at[NCHUNK - 1]], buf0, sem0).wait()
    pltpu.make_async_copy(hp.at[src2.at[NCHUNK - 1]], buf1, sem1).wait()
    plsc.subcore_barrier()
    pltpu.sync_copy(agg_sh.at[pl.ds(r0, SPAN)], out.at[cid, pl.ds(r0, SPAN)])


_RB = 1000


def _prep_body(x_ref, wi_ref, wr_ref, b_ref, d0_ref, d1_ref,
               hp_ref, rootb_ref, dinv_ref):
    x = x_ref[...]
    deg = d0_ref[0, :, 0:1] + d1_ref[0, :, 0:1]
    dinv = jnp.where(deg > 0, lax.rsqrt(deg), 0.0)
    h = jnp.dot(x, wi_ref[...], preferred_element_type=jnp.float32)
    hp_ref[...] = h * dinv
    rootb_ref[...] = (
        jnp.dot(x, wr_ref[...], preferred_element_type=jnp.float32) + b_ref[...]
    )
    dinv_ref[...] = dinv


def _prep(x, wi, wr, b2, degp):
    grid = (N // _RB,)
    return pl.pallas_call(
        _prep_body,
        grid=grid,
        in_specs=[
            pl.BlockSpec((_RB, F_IN), lambda i: (i, 0)),
            pl.BlockSpec((F_IN, F_OUT), lambda i: (0, 0)),
            pl.BlockSpec((F_IN, F_OUT), lambda i: (0, 0)),
            pl.BlockSpec((1, F_OUT), lambda i: (0, 0)),
            pl.BlockSpec((1, _RB, DEG_W), lambda i: (0, i, 0)),
            pl.BlockSpec((1, _RB, DEG_W), lambda i: (1, i, 0)),
        ],
        out_specs=[
            pl.BlockSpec((_RB, F_OUT), lambda i: (i, 0)),
            pl.BlockSpec((_RB, F_OUT), lambda i: (i, 0)),
            pl.BlockSpec((_RB, 1), lambda i: (i, 0)),
        ],
        out_shape=[
            jax.ShapeDtypeStruct((N, F_OUT), jnp.float32),
            jax.ShapeDtypeStruct((N, F_OUT), jnp.float32),
            jax.ShapeDtypeStruct((N, 1), jnp.float32),
        ],
    )(x, wi, wr, b2, degp, degp)


def _final_body(p0_ref, p1_ref, dinv_ref, rootb_ref, o_ref):
    agg = p0_ref[0] + p1_ref[0]
    o_ref[...] = jnp.maximum(dinv_ref[...] * agg + rootb_ref[...], 0.0)


def _final(p, dinv, rootb):
    grid = (N // _RB,)
    return pl.pallas_call(
        _final_body,
        grid=grid,
        in_specs=[
            pl.BlockSpec((1, _RB, F_OUT), lambda i: (0, i, 0)),
            pl.BlockSpec((1, _RB, F_OUT), lambda i: (1, i, 0)),
            pl.BlockSpec((_RB, 1), lambda i: (i, 0)),
            pl.BlockSpec((_RB, F_OUT), lambda i: (i, 0)),
        ],
        out_specs=pl.BlockSpec((_RB, F_OUT), lambda i: (i, 0)),
        out_shape=jax.ShapeDtypeStruct((N, F_OUT), jnp.float32),
    )(p, p, dinv, rootb)


def kernel(x, edge_index, W_init, W_root, bias):
    src3 = edge_index[0].reshape(NW, NCHUNK, CH)
    dst3 = edge_index[1].reshape(NW, NCHUNK, CH)
    ones8 = jnp.ones((CH, DEG_W), jnp.float32)
    zeros8 = jnp.zeros((DEG_SPAN, DEG_W), jnp.float32)
    degp = _deg_kernel(dst3, ones8, zeros8)
    hp, rootb, dinv = _prep(x, W_init, W_root, bias.reshape(1, F_OUT), degp)
    p = _agg_kernel(hp, src3, dst3)
    return _final(p, dinv, rootb)


# speedup vs baseline: 42.0625x; 1.0826x over previous
import functools

import jax
import jax.numpy as jnp
from jax import lax
from jax.experimental import pallas as pl
from jax.experimental.pallas import tpu as pltpu
from jax.experimental.pallas import tpu_sc as plsc

N = 10000
E = 320000
F_IN = 128
F_OUT = 64

NC = 2
NS = 16
NW = NC * NS
EPW = E // NW
CH = 125
NCHUNK = EPW // CH
SPAN = 632
N_PAD = SPAN * NS
DEG_PAD = 10240
DEG_SPAN = DEG_PAD // NS
ZROWS = 80

_MESH = plsc.VectorSubcoreMesh(core_axis_name="c", subcore_axis_name="s")
_SC_PARAMS = pltpu.CompilerParams(use_tc_tiling_on_sc=False)


DEG_W = 8


@functools.partial(
    pl.kernel,
    out_type=jax.ShapeDtypeStruct((NC, DEG_PAD, DEG_W), jnp.float32),
    mesh=_MESH,
    compiler_params=_SC_PARAMS,
    scratch_types=[
        pltpu.VMEM((NCHUNK, CH), jnp.int32),
        pltpu.VMEM((CH, DEG_W), jnp.float32),
        pltpu.VMEM((DEG_SPAN, DEG_W), jnp.float32),
        pltpu.VMEM_SHARED((DEG_PAD, DEG_W), jnp.float32),
        pltpu.SemaphoreType.DMA,
    ],
)
def _deg_kernel(dst3, ones8, zeros8, degp, dst2, ones_v, zb, deg_sh, dsem):
    cid = lax.axis_index("c")
    sid = lax.axis_index("s")
    wid = cid * NS + sid

    pltpu.sync_copy(zeros8, zb)
    pltpu.sync_copy(ones8, ones_v)
    pltpu.sync_copy(zb, deg_sh.at[pl.ds(sid * DEG_SPAN, DEG_SPAN)])
    pltpu.sync_copy(dst3.at[wid], dst2)
    plsc.subcore_barrier()

    @pl.loop(0, NCHUNK)
    def _accum(j):
        pltpu.async_copy(ones_v, deg_sh.at[dst2.at[j]], dsem, add=True)

    @pl.loop(0, NCHUNK)
    def _drain(j):
        pltpu.make_async_copy(ones_v, deg_sh.at[dst2.at[j]], dsem).wait()

    plsc.subcore_barrier()
    pltpu.sync_copy(
        deg_sh.at[pl.ds(sid * DEG_SPAN, DEG_SPAN)],
        degp.at[cid, pl.ds(sid * DEG_SPAN, DEG_SPAN)],
    )


NBUF = 8
DEPTH = 4


@functools.partial(
    pl.kernel,
    out_type=jax.ShapeDtypeStruct((NC, N_PAD, F_OUT), jnp.float32),
    mesh=_MESH,
    compiler_params=_SC_PARAMS,
    scratch_types=[
        pltpu.VMEM((NCHUNK, CH), jnp.int32),
        pltpu.VMEM((NCHUNK, CH), jnp.int32),
    ]
    + [pltpu.VMEM((CH, F_OUT), jnp.float32)] * NBUF
    + [
        pltpu.VMEM((ZROWS, F_OUT), jnp.float32),
        pltpu.VMEM_SHARED((N_PAD, F_OUT), jnp.float32),
    ]
    + [pltpu.SemaphoreType.DMA] * (2 * NBUF),
)
def _agg_kernel(hp, src3, dst3, out, src2, dst2, *rest):
    bufs = rest[:NBUF]
    zbuf = rest[NBUF]
    agg_sh = rest[NBUF + 1]
    gsem = rest[NBUF + 2:NBUF + 2 + NBUF]
    ssem = rest[NBUF + 2 + NBUF:]
    cid = lax.axis_index("c")
    sid = lax.axis_index("s")
    wid = cid * NS + sid

    @pl.loop(0, ZROWS)
    def _zero(r):
        for c in range(F_OUT // 16):
            zbuf[r, pl.ds(c * 16, 16)] = jnp.zeros((16,), jnp.float32)

    r0 = sid * SPAN
    for t in range(SPAN // ZROWS):
        pltpu.sync_copy(zbuf, agg_sh.at[pl.ds(r0 + t * ZROWS, ZROWS)])
    rem = SPAN - (SPAN // ZROWS) * ZROWS
    pltpu.sync_copy(
        zbuf.at[pl.ds(0, rem)],
        agg_sh.at[pl.ds(r0 + SPAN - rem, rem)],
    )
    pltpu.sync_copy(src3.at[wid], src2)
    pltpu.sync_copy(dst3.at[wid], dst2)
    plsc.subcore_barrier()

    def wait_g(k, b):
        pltpu.make_async_copy(hp.at[src2.at[k]], bufs[b], gsem[b]).wait()

    def fire_s(k, b):
        pltpu.async_copy(bufs[b], agg_sh.at[dst2.at[k]], ssem[b], add=True)

    def wait_s(k, b):
        pltpu.make_async_copy(bufs[b], agg_sh.at[dst2.at[k]], ssem[b]).wait()

    for k in range(DEPTH):
        pltpu.async_copy(hp.at[src2.at[k]], bufs[k % NBUF], gsem[k % NBUF])
    for k in range(DEPTH):
        bb = k % NBUF
        wait_g(k, bb)
        fire_s(k, bb)
        b4 = (k + DEPTH) % NBUF
        pltpu.async_copy(hp.at[src2.at[k + DEPTH]], bufs[b4], gsem[b4])

    @pl.loop(DEPTH, NCHUNK - DEPTH, step=NBUF)
    def _edges(j):
        for u in range(NBUF):
            k = j + u
            bb = (DEPTH + u) % NBUF
            b4 = u % NBUF
            wait_g(k, bb)
            fire_s(k, bb)
            wait_s(k - DEPTH, b4)
            pltpu.async_copy(hp.at[src2.at[k + DEPTH]], bufs[b4], gsem[b4])

    for k in range(NCHUNK - DEPTH, NCHUNK):
        bb = k % NBUF
        wait_g(k, bb)
        fire_s(k, bb)
        wait_s(k - DEPTH, (k + DEPTH) % NBUF)
    for k in range(NCHUNK - DEPTH, NCHUNK):
        wait_s(k, k % NBUF)
    plsc.subcore_barrier()
    pltpu.sync_copy(agg_sh.at[pl.ds(r0, SPAN)], out.at[cid, pl.ds(r0, SPAN)])


_RB = 1000


def _mm_body(x_ref, wi_ref, wr_ref, b_ref, h_ref, rootb_ref):
    x = x_ref[...]
    h_ref[...] = jnp.dot(x, wi_ref[...], preferred_element_type=jnp.float32)
    rootb_ref[...] = (
        jnp.dot(x, wr_ref[...], preferred_element_type=jnp.float32) + b_ref[...]
    )


def _mm(x, wi, wr, b2):
    grid = (N // _RB,)
    return pl.pallas_call(
        _mm_body,
        grid=grid,
        in_specs=[
            pl.BlockSpec((_RB, F_IN), lambda i: (i, 0)),
            pl.BlockSpec((F_IN, F_OUT), lambda i: (0, 0)),
            pl.BlockSpec((F_IN, F_OUT), lambda i: (0, 0)),
            pl.BlockSpec((1, F_OUT), lambda i: (0, 0)),
        ],
        out_specs=[
            pl.BlockSpec((_RB, F_OUT), lambda i: (i, 0)),
            pl.BlockSpec((_RB, F_OUT), lambda i: (i, 0)),
        ],
        out_shape=[
            jax.ShapeDtypeStruct((N, F_OUT), jnp.float32),
            jax.ShapeDtypeStruct((N, F_OUT), jnp.float32),
        ],
    )(x, wi, wr, b2)


def _scale_body(h_ref, d0_ref, d1_ref, hp_ref, dinv_ref):
    deg = d0_ref[0, :, 0:1] + d1_ref[0, :, 0:1]
    dinv = jnp.where(deg > 0, lax.rsqrt(deg), 0.0)
    hp_ref[...] = h_ref[...] * dinv
    dinv_ref[...] = dinv


def _scale(h, degp):
    grid = (N // _RB,)
    return pl.pallas_call(
        _scale_body,
        grid=grid,
        in_specs=[
            pl.BlockSpec((_RB, F_OUT), lambda i: (i, 0)),
            pl.BlockSpec((1, _RB, DEG_W), lambda i: (0, i, 0)),
            pl.BlockSpec((1, _RB, DEG_W), lambda i: (1, i, 0)),
        ],
        out_specs=[
            pl.BlockSpec((_RB, F_OUT), lambda i: (i, 0)),
            pl.BlockSpec((_RB, 1), lambda i: (i, 0)),
        ],
        out_shape=[
            jax.ShapeDtypeStruct((N, F_OUT), jnp.float32),
            jax.ShapeDtypeStruct((N, 1), jnp.float32),
        ],
    )(h, degp, degp)


def _final_body(p0_ref, p1_ref, dinv_ref, rootb_ref, o_ref):
    agg = p0_ref[0] + p1_ref[0]
    o_ref[...] = jnp.maximum(dinv_ref[...] * agg + rootb_ref[...], 0.0)


def _final(p, dinv, rootb):
    grid = (N // _RB,)
    return pl.pallas_call(
        _final_body,
        grid=grid,
        in_specs=[
            pl.BlockSpec((1, _RB, F_OUT), lambda i: (0, i, 0)),
            pl.BlockSpec((1, _RB, F_OUT), lambda i: (1, i, 0)),
            pl.BlockSpec((_RB, 1), lambda i: (i, 0)),
            pl.BlockSpec((_RB, F_OUT), lambda i: (i, 0)),
        ],
        out_specs=pl.BlockSpec((_RB, F_OUT), lambda i: (i, 0)),
        out_shape=jax.ShapeDtypeStruct((N, F_OUT), jnp.float32),
    )(p, p, dinv, rootb)


def kernel(x, edge_index, W_init, W_root, bias):
    src3 = edge_index[0].reshape(NW, NCHUNK, CH)
    dst3 = edge_index[1].reshape(NW, NCHUNK, CH)
    ones8 = jnp.ones((CH, DEG_W), jnp.float32)
    zeros8 = jnp.zeros((DEG_SPAN, DEG_W), jnp.float32)
    degp = _deg_kernel(dst3, ones8, zeros8)
    h, rootb = _mm(x, W_init, W_root, bias.reshape(1, F_OUT))
    hp, dinv = _scale(h, degp)
    p = _agg_kernel(hp, src3, dst3)
    return _final(p, dinv, rootb)
